# Initial kernel scaffold; baseline (speedup 1.0000x reference)
import jax
import jax.numpy as jnp
from jax.experimental import pallas as pl


def kernel(coord, feat, offset, params):
    raise NotImplementedError("write your pallas kernel here")


# speedup vs baseline: 4.8674x; 4.8674x over previous
import functools

import jax
import jax.numpy as jnp
from jax import lax
from jax.experimental import pallas as pl

_PLANES = [32, 64, 128, 256, 512]
_STRIDE = [1, 4, 4, 4, 4]
_NSAMPLE = [8, 16, 16, 16, 16]
_EPS = 1e-5


def _pow2_block(n, cap):
    b = 1
    while b * 2 <= min(n, cap) and n % (b * 2) == 0:
        b *= 2
    return b


def _stats(y):
    m = y.mean(0)
    v = y.var(0)
    return m, jnp.sqrt(v + _EPS)


def _mm_kernel(x_ref, w_ref, b_ref, m_ref, d_ref, g_ref, bb_ref, y_ref, *,
               pre_bn, relu_in, bias, pret):
    x = x_ref[...]
    if pre_bn:
        x = g_ref[...] * (x - m_ref[...]) / d_ref[...] + bb_ref[...]
        if relu_in:
            x = jnp.maximum(x, 0.0)
    dims = (((1,), (0,)), ((), ())) if pret else (((1,), (1,)), ((), ()))
    y = lax.dot_general(x, w_ref[...], dims,
                        precision=lax.Precision.DEFAULT,
                        preferred_element_type=jnp.float32)
    if bias:
        y = y + b_ref[...]
    y_ref[...] = y


def _mm_call(x, lin, bn=None, relu_in=True, pret=False):
    n, cin = x.shape
    w = lin["W"].T if pret else lin["W"]
    cout = w.shape[1] if pret else w.shape[0]
    bias = "b" in lin
    b = lin["b"] if bias else jnp.zeros((cout,), jnp.float32)
    if bn is None:
        m = jnp.zeros((cin,), jnp.float32)
        d = jnp.ones((cin,), jnp.float32)
        g = jnp.ones((cin,), jnp.float32)
        bb = jnp.zeros((cin,), jnp.float32)
    else:
        m, d, g, bb = bn
    bnn = _pow2_block(n, max(8, (1 << 18) // max(cin, cout, 128)))
    wspec = (cin, cout) if pret else (cout, cin)
    return pl.pallas_call(
        functools.partial(_mm_kernel, pre_bn=bn is not None, relu_in=relu_in,
                          bias=bias, pret=pret),
        grid=(n // bnn,),
        in_specs=[
            pl.BlockSpec((bnn, cin), lambda i: (i, 0)),
            pl.BlockSpec(wspec, lambda i: (0, 0)),
            pl.BlockSpec((1, cout), lambda i: (0, 0)),
            pl.BlockSpec((1, cin), lambda i: (0, 0)),
            pl.BlockSpec((1, cin), lambda i: (0, 0)),
            pl.BlockSpec((1, cin), lambda i: (0, 0)),
            pl.BlockSpec((1, cin), lambda i: (0, 0)),
        ],
        out_specs=pl.BlockSpec((bnn, cout), lambda i: (i, 0)),
        out_shape=jax.ShapeDtypeStruct((n, cout), jnp.float32),
    )(x, w, b.reshape(1, cout), m.reshape(1, cin), d.reshape(1, cin),
      g.reshape(1, cin), bb.reshape(1, cin))


def _maxk_kernel(y_ref, m_ref, d_ref, g_ref, b_ref, o_ref):
    h = g_ref[...] * (y_ref[...] - m_ref[...]) / d_ref[...] + b_ref[...]
    h = jnp.maximum(h, 0.0)
    o_ref[...] = jnp.max(h, axis=1)


def _maxk_call(y3, m, d, g, b):
    n, k, c = y3.shape
    bn = _pow2_block(n, max(8, (1 << 18) // (k * max(c, 128))))
    r = lambda v: v.reshape(1, 1, c)
    return pl.pallas_call(
        _maxk_kernel,
        grid=(n // bn,),
        in_specs=[pl.BlockSpec((bn, k, c), lambda i: (i, 0, 0))] +
                 [pl.BlockSpec((1, 1, c), lambda i: (0, 0, 0))] * 4,
        out_specs=pl.BlockSpec((bn, c), lambda i: (i, 0)),
        out_shape=jax.ShapeDtypeStruct((n, c), jnp.float32),
    )(y3, r(m), r(d), r(g), r(b))


def _bnaddrelu_kernel(y_ref, m_ref, d_ref, g_ref, b_ref, r_ref, o_ref):
    h = g_ref[...] * (y_ref[...] - m_ref[...]) / d_ref[...] + b_ref[...]
    o_ref[...] = jnp.maximum(h + r_ref[...], 0.0)


def _bnaddrelu_call(y, m, d, g, b, res):
    n, c = y.shape
    bn = _pow2_block(n, max(8, (1 << 18) // max(c, 128)))
    r = lambda v: v.reshape(1, c)
    return pl.pallas_call(
        _bnaddrelu_kernel,
        grid=(n // bn,),
        in_specs=[pl.BlockSpec((bn, c), lambda i: (i, 0))] +
                 [pl.BlockSpec((1, c), lambda i: (0, 0))] * 4 +
                 [pl.BlockSpec((bn, c), lambda i: (i, 0))],
        out_specs=pl.BlockSpec((bn, c), lambda i: (i, 0)),
        out_shape=jax.ShapeDtypeStruct((n, c), jnp.float32),
    )(y, r(m), r(d), r(g), r(b), res)


def _bnrelu_kernel(y_ref, m_ref, d_ref, g_ref, b_ref, o_ref):
    h = g_ref[...] * (y_ref[...] - m_ref[...]) / d_ref[...] + b_ref[...]
    o_ref[...] = jnp.maximum(h, 0.0)


def _bnrelu_call(y, m, d, g, b):
    n, c = y.shape
    bn = _pow2_block(n, max(8, (1 << 18) // max(c, 128)))
    r = lambda v: v.reshape(1, c)
    return pl.pallas_call(
        _bnrelu_kernel,
        grid=(n // bn,),
        in_specs=[pl.BlockSpec((bn, c), lambda i: (i, 0))] +
                 [pl.BlockSpec((1, c), lambda i: (0, 0))] * 4,
        out_specs=pl.BlockSpec((bn, c), lambda i: (i, 0)),
        out_shape=jax.ShapeDtypeStruct((n, c), jnp.float32),
    )(y, r(m), r(d), r(g), r(b))


def _knn_kernel(q_ref, bt_ref, idx_ref, *, k, nb):
    q = q_ref[...]
    qx, qy, qz = q[:, 0:1], q[:, 1:2], q[:, 2:3]
    bx = bt_ref[0:1, :]
    by = bt_ref[1:2, :]
    bz = bt_ref[2:3, :]
    qq = (qx * qx + qz * qz) + qy * qy
    bb = (bx * bx + bz * bz) + by * by
    dot = lax.dot_general(q, bt_ref[...], (((1,), (0,)), ((), ())),
                          precision=lax.Precision.DEFAULT,
                          preferred_element_type=jnp.float32)
    d = (qq - 2.0 * dot) + bb
    bq = q.shape[0]
    iota = lax.broadcasted_iota(jnp.int32, (bq, nb), 1)
    idx_cols = []
    for _ in range(k):
        rowmin = jnp.min(d, axis=1, keepdims=True)
        rowidx = jnp.min(jnp.where(d == rowmin, iota, nb), axis=1,
                         keepdims=True)
        idx_cols.append(rowidx)
        d = jnp.where(iota == rowidx, jnp.float32(3e38), d)
    idx_ref[...] = jnp.concatenate(idx_cols, axis=1)


def _knn_call(q, base, k):
    nq = q.shape[0]
    nb = base.shape[0]
    bq = _pow2_block(nq, max(8, (1 << 20) // nb))
    return pl.pallas_call(
        functools.partial(_knn_kernel, k=k, nb=nb),
        grid=(nq // bq,),
        in_specs=[
            pl.BlockSpec((bq, 3), lambda i: (i, 0)),
            pl.BlockSpec((3, nb), lambda i: (0, 0)),
        ],
        out_specs=pl.BlockSpec((bq, k), lambda i: (i, 0)),
        out_shape=jax.ShapeDtypeStruct((nq, k), jnp.int32),
    )(q, base.T)


def _fps_kernel(pt_ref, idx_ref, *, n, m):
    px = pt_ref[0:1, :]
    py = pt_ref[1:2, :]
    pz = pt_ref[2:3, :]
    iota = lax.broadcasted_iota(jnp.int32, (1, n), 1)
    iota_m = lax.broadcasted_iota(jnp.int32, (1, m), 1)

    def body(i, carry):
        dists, last, sel = carry
        lx = jnp.sum(jnp.where(iota == last, px, 0.0))
        ly = jnp.sum(jnp.where(iota == last, py, 0.0))
        lz = jnp.sum(jnp.where(iota == last, pz, 0.0))
        dx, dy, dz = px - lx, py - ly, pz - lz
        d = (dx * dx + dy * dy) + dz * dz
        dists = jnp.minimum(dists, d)
        mx = jnp.max(dists)
        nxt = jnp.min(jnp.where(dists == mx, iota, n)).astype(jnp.int32)
        sel = jnp.where(iota_m == i, nxt, sel)
        return dists, nxt, sel

    dists0 = jnp.full((1, n), 1e10, jnp.float32)
    sel0 = jnp.zeros((1, m), jnp.int32)
    _, _, sel = lax.fori_loop(1, m, body, (dists0, jnp.int32(0), sel0))
    idx_ref[...] = sel


def _fps_call(p, m):
    n = p.shape[0]
    out = pl.pallas_call(
        functools.partial(_fps_kernel, n=n, m=m),
        out_shape=jax.ShapeDtypeStruct((1, m), jnp.int32),
    )(p.T)
    return out.reshape(m)


def _wsum_kernel(ya_ref, ma_ref, da_ref, ga_ref, ba_ref,
                 bg_ref, mb_ref, db_ref, gb_ref, bb_ref,
                 p1_ref, nb_ref, o_ref):
    a = ga_ref[...] * (ya_ref[...] - ma_ref[...]) / da_ref[...] + ba_ref[...]
    a = jnp.maximum(a, 0.0)
    h = gb_ref[...] * (bg_ref[...] - mb_ref[...]) / db_ref[...] + bb_ref[...]
    h = jnp.maximum(h, 0.0)
    p1 = p1_ref[...]
    dx = p1[:, None, 0] - nb_ref[:, :, 0]
    dy = p1[:, None, 1] - nb_ref[:, :, 1]
    dz = p1[:, None, 2] - nb_ref[:, :, 2]
    dsq = (dx * dx + dz * dz) + dy * dy
    dd = jnp.sqrt(dsq + 1e-12)
    w = 1.0 / (dd + 1e-8)
    ws = (w[:, 0] + w[:, 2]) + w[:, 1]
    wn = w / ws[:, None]
    t = h * wn[:, :, None]
    o_ref[...] = a + ((t[:, 0, :] + t[:, 1, :]) + t[:, 2, :])


def _wsum_call(ya, bna, bg, bnb, p1, nb):
    n, kk, c = bg.shape
    bn = _pow2_block(n, max(8, (1 << 17) // (kk * max(c, 128))))
    r2 = lambda v: v.reshape(1, c)
    r3 = lambda v: v.reshape(1, 1, c)
    ma, da, ga, ba = bna
    mb, db, gb, bb = bnb
    return pl.pallas_call(
        _wsum_kernel,
        grid=(n // bn,),
        in_specs=[pl.BlockSpec((bn, c), lambda i: (i, 0))] +
                 [pl.BlockSpec((1, c), lambda i: (0, 0))] * 4 +
                 [pl.BlockSpec((bn, kk, c), lambda i: (i, 0, 0))] +
                 [pl.BlockSpec((1, 1, c), lambda i: (0, 0, 0))] * 4 +
                 [pl.BlockSpec((bn, 3), lambda i: (i, 0)),
                  pl.BlockSpec((bn, kk, 3), lambda i: (i, 0, 0))],
        out_specs=pl.BlockSpec((bn, c), lambda i: (i, 0)),
        out_shape=jax.ShapeDtypeStruct((n, c), jnp.float32),
    )(ya, r2(ma), r2(da), r2(ga), r2(ba),
      bg, r3(mb), r3(db), r3(gb), r3(bb), p1, nb)


def _bottleneck(p, x, k, prm):
    n = x.shape[0]
    identity = x
    y1 = _mm_call(x, prm["lin1"])
    m1, d1 = _stats(y1)
    bn1 = (m1, d1, prm["bn1"]["g"], prm["bn1"]["b"])
    idx = _knn_call(p, p, k)
    g = jnp.take(y1, idx.reshape(-1), axis=0)
    yl1 = _mm_call(g, prm["la_l1"], bn=bn1)
    ml1, dl1 = _stats(yl1)
    yl2 = _mm_call(yl1, prm["la_l2"],
                   bn=(ml1, dl1, prm["la_bn1"]["g"], prm["la_bn1"]["b"]))
    ml2, dl2 = _stats(yl2)
    la = _maxk_call(yl2.reshape(n, k, -1), ml2, dl2,
                    prm["la_bn2"]["g"], prm["la_bn2"]["b"])
    m2, d2 = _stats(la)
    y3 = _mm_call(la, prm["lin3"],
                  bn=(m2, d2, prm["bn2"]["g"], prm["bn2"]["b"]))
    m3, d3 = _stats(y3)
    return _bnaddrelu_call(y3, m3, d3, prm["bn3"]["g"], prm["bn3"]["b"],
                           identity)


def _transition_down(p, x, stride, k, prm):
    if stride == 1:
        y = _mm_call(x, prm["lin"])
        m, d = _stats(y)
        return p, _bnrelu_call(y, m, d, prm["bn"]["g"], prm["bn"]["b"])
    n = p.shape[0]
    m_pts = n // stride
    fidx = _fps_call(p, m_pts)
    new_p = jnp.take(p, fidx, axis=0)
    nidx = _knn_call(new_p, p, k)
    rel = jnp.take(p, nidx.reshape(-1), axis=0).reshape(m_pts, k, 3) \
        - new_p[:, None, :]
    xg = jnp.take(x, nidx.reshape(-1), axis=0).reshape(m_pts, k, -1)
    g = jnp.concatenate([rel, xg], axis=-1).reshape(m_pts * k, -1)
    y = _mm_call(g, prm["lin"])
    ms, ds = _stats(y)
    h = _maxk_call(y.reshape(m_pts, k, -1), ms, ds,
                   prm["bn"]["g"], prm["bn"]["b"])
    return new_p, h


def _transition_up_head(x, prm):
    n, c = x.shape
    xm = jnp.broadcast_to(jnp.mean(x, axis=0, keepdims=True), (8, c))
    y2 = _mm_call(xm, prm["l2"])
    gv = jax.nn.relu(y2[0:1])
    xc = jnp.concatenate([x, jnp.broadcast_to(gv, (n, c))], axis=1)
    y1 = _mm_call(xc, prm["l1"])
    m1, d1 = _stats(y1)
    return _bnrelu_call(y1, m1, d1, prm["l1bn"]["g"], prm["l1bn"]["b"])


def _transition_up(p1, x1, p2, x2, prm):
    n1 = x1.shape[0]
    ya = _mm_call(x1, prm["l1"])
    ma, da = _stats(ya)
    yb = _mm_call(x2, prm["l2"], pret=(x2.shape[0], x2.shape[1]) == (128, 256))
    mb, db = _stats(yb)
    idx = _knn_call(p1, p2, 3)
    flat = idx.reshape(-1)
    nb = jnp.take(p2, flat, axis=0).reshape(n1, 3, 3)
    bg = jnp.take(yb, flat, axis=0).reshape(n1, 3, -1)
    return _wsum_call(ya, (ma, da, prm["l1bn"]["g"], prm["l1bn"]["b"]),
                      bg, (mb, db, prm["l2bn"]["g"], prm["l2bn"]["b"]),
                      p1, nb)


def kernel(coord, feat, offset, params):
    del offset
    p, x = coord, feat
    ps, xs = [], []
    for i in range(5):
        prm = params["enc"][i]
        p, x = _transition_down(p, x, _STRIDE[i], _NSAMPLE[i], prm["td"])
        x = _bottleneck(p, x, _NSAMPLE[i], prm["blk"])
        ps.append(p)
        xs.append(x)
    x = _transition_up_head(xs[4], params["dec"][0]["tu"])
    xs[4] = _bottleneck(ps[4], x, _NSAMPLE[4], params["dec"][0]["blk"])
    for j, i in enumerate(range(3, -1, -1)):
        prm = params["dec"][j + 1]
        x = _transition_up(ps[i], xs[i], ps[i + 1], xs[i + 1], prm["tu"])
        xs[i] = _bottleneck(ps[i], x, _NSAMPLE[i], prm["blk"])
    cls = params["cls"]
    yh = _mm_call(xs[0], cls["l1"])
    mh, dh = _stats(yh)
    return _mm_call(yh, cls["l2"],
                    bn=(mh, dh, cls["bn"]["g"], cls["bn"]["b"]))
